# baseline (speedup 1.0000x reference)
import functools

import jax
import jax.numpy as jnp
from jax import lax
from jax.experimental import pallas as pl
from jax.experimental.pallas import tpu as pltpu
from jax.experimental.pallas import tpu_sc as plsc

N = 10000
D = 256
E = 160000
K = N // 2
TILE = 1024
NP = 10240
GRID = NP // TILE
KP = 5120

NC, NS = 2, 16
NW = NC * NS
NODES_W = NP // NW
CH = 64
NCH = NODES_W // CH
EW = E // NW
EWP = 5008
NEG_INF = float("-inf")


def _score_body(x_ref, w_ref, srow_ref):
    i = pl.program_id(0)
    x = x_ref[...]
    w = w_ref[...]
    row = lax.dot_general(w, x, (((1,), (1,)), ((), ())),
                          preferred_element_type=jnp.float32)
    cidx = i * TILE + lax.broadcasted_iota(jnp.int32, (1, TILE), 1)
    srow_ref[...] = jnp.where(cidx < N, row, NEG_INF)


def _rank_body(scol_ref, srow_ref, nid_ref):
    i = pl.program_id(0)
    si = scol_ref[...]
    iidx = i * TILE + lax.broadcasted_iota(jnp.int32, (TILE, 1), 0)
    acc = jnp.zeros((TILE, 1), jnp.float32)
    for t in range(GRID):
        sj = srow_ref[:, t * TILE:(t + 1) * TILE]
        jidx = t * TILE + lax.broadcasted_iota(jnp.int32, (1, TILE), 1)
        m = (sj > si) | ((sj == si) & (jidx < iidx))
        acc = acc + jnp.sum(m.astype(jnp.float32), axis=1, keepdims=True)
    rank = acc.astype(jnp.int32)
    nid_ref[...] = jnp.where(rank < K, rank, -1)


def _sc_body(newid_hbm, feats_hbm, esrc_hbm, edst_hbm,
             idx_hbm, h_hbm, osrc_hbm, odst_hbm,
             table_v, src_v, dst_v, rsrc_v, rdst_v,
             gidx_v, tgt_v, vals16_v, rows_v, sem):
    cid = lax.axis_index("c")
    sid = lax.axis_index("s")
    wid = sid * NC + cid

    pltpu.sync_copy(newid_hbm, table_v)

    ebase = wid * EW
    zeros16 = jnp.zeros((16,), jnp.int32)
    src_v[pl.ds(EWP - 16, 16)] = zeros16
    dst_v[pl.ds(EWP - 16, 16)] = zeros16
    pltpu.sync_copy(esrc_hbm.at[pl.ds(ebase, EW)], src_v.at[pl.ds(0, EW)])
    pltpu.sync_copy(edst_hbm.at[pl.ds(ebase, EW)], dst_v.at[pl.ds(0, EW)])

    def ebody(t, carry):
        off = pl.multiple_of(t * 16, 16)
        s16 = src_v[pl.ds(off, 16)]
        d16 = dst_v[pl.ds(off, 16)]
        a = plsc.load_gather(table_v, [s16])
        b2 = plsc.load_gather(table_v, [d16])
        valid = (a >= 0) & (b2 >= 0)
        rsrc_v[pl.ds(off, 16)] = jnp.where(valid, a, -1)
        rdst_v[pl.ds(off, 16)] = jnp.where(valid, b2, -1)
        return carry

    lax.fori_loop(0, EWP // 16, ebody, 0)
    pltpu.sync_copy(rsrc_v.at[pl.ds(0, EW)], osrc_hbm.at[pl.ds(ebase, EW)])
    pltpu.sync_copy(rdst_v.at[pl.ds(0, EW)], odst_hbm.at[pl.ds(ebase, EW)])

    nbase = wid * NODES_W
    dummy = K + wid
    lane = lax.iota(jnp.int32, 16)
    zlane = jnp.zeros((16,), jnp.int32)
    for ci in range(NCH):
        base = nbase + ci * CH
        for t in range(CH // 16):
            boff = pl.multiple_of(base + t * 16, 16)
            nid = table_v[pl.ds(boff, 16)]
            lidx = boff + lane
            sel = nid >= 0
            gidx_v[ci, pl.ds(t * 16, 16)] = jnp.minimum(lidx, N - 1)
            tgt_v[ci, pl.ds(t * 16, 16)] = jnp.where(sel, nid, dummy)
            plsc.store_scatter(vals16_v, [t * 16 + lane, zlane], lidx)
        pltpu.async_copy(feats_hbm.at[gidx_v.at[ci]], rows_v, sem).wait()
        pltpu.async_copy(rows_v, h_hbm.at[tgt_v.at[ci]], sem).wait()
        pltpu.async_copy(vals16_v, idx_hbm.at[tgt_v.at[ci]], sem).wait()


@jax.jit
def kernel(node_features, edge_index, W, b):
    del b
    srow = pl.pallas_call(
        _score_body,
        grid=(GRID,),
        in_specs=[
            pl.BlockSpec((TILE, D), lambda i: (i, 0)),
            pl.BlockSpec((1, D), lambda i: (0, 0)),
        ],
        out_specs=pl.BlockSpec((1, TILE), lambda i: (0, i)),
        out_shape=jax.ShapeDtypeStruct((1, NP), jnp.float32),
    )(node_features, W)
    scol = jnp.reshape(srow, (NP, 1))

    new_id2 = pl.pallas_call(
        _rank_body,
        grid=(GRID,),
        in_specs=[
            pl.BlockSpec((TILE, 1), lambda i: (i, 0)),
            pl.BlockSpec((1, NP), lambda i: (0, 0)),
        ],
        out_specs=pl.BlockSpec((TILE, 1), lambda i: (i, 0)),
        out_shape=jax.ShapeDtypeStruct((NP, 1), jnp.int32),
    )(scol, srow)
    new_id = jnp.reshape(new_id2, (NP,))

    sc = pl.kernel(
        _sc_body,
        out_type=(
            jax.ShapeDtypeStruct((KP, 128), jnp.int32),
            jax.ShapeDtypeStruct((KP, D), jnp.float32),
            jax.ShapeDtypeStruct((E,), jnp.int32),
            jax.ShapeDtypeStruct((E,), jnp.int32),
        ),
        mesh=plsc.VectorSubcoreMesh(
            core_axis_name="c", subcore_axis_name="s",
            num_cores=NC, num_subcores=NS),
        compiler_params=pltpu.CompilerParams(needs_layout_passes=False),
        scratch_types=[
            pltpu.VMEM((NP,), jnp.int32),
            pltpu.VMEM((EWP,), jnp.int32),
            pltpu.VMEM((EWP,), jnp.int32),
            pltpu.VMEM((EWP,), jnp.int32),
            pltpu.VMEM((EWP,), jnp.int32),
            pltpu.VMEM((NCH, CH), jnp.int32),
            pltpu.VMEM((NCH, CH), jnp.int32),
            pltpu.VMEM((CH, 128), jnp.int32),
            pltpu.VMEM((CH, D), jnp.float32),
            pltpu.SemaphoreType.DMA,
        ],
    )
    idx_pad, h_pad, osrc, odst = sc(new_id, node_features,
                                    edge_index[0], edge_index[1])
    return (h_pad[:K], idx_pad[:K, 0], jnp.stack([osrc, odst]))


# speedup vs baseline: 1.3782x; 1.3782x over previous
import functools

import jax
import jax.numpy as jnp
from jax import lax
from jax.experimental import pallas as pl
from jax.experimental.pallas import tpu as pltpu
from jax.experimental.pallas import tpu_sc as plsc

N = 10000
D = 256
E = 160000
K = N // 2
TILE = 1024
NP = 10240
GRID = NP // TILE
KP = 5120

NC, NS = 2, 16
NW = NC * NS
NODES_W = NP // NW
CH = 64
NCH = NODES_W // CH
EW = E // NW
EWP = 5008
NEG_INF = float("-inf")


def _score_body(x_ref, w_ref, srow_ref):
    i = pl.program_id(0)
    x = x_ref[...]
    w = w_ref[...]
    row = lax.dot_general(w, x, (((1,), (1,)), ((), ())),
                          preferred_element_type=jnp.float32)
    cidx = i * TILE + lax.broadcasted_iota(jnp.int32, (1, TILE), 1)
    srow_ref[...] = jnp.where(cidx < N, row, NEG_INF)


def _rank_body(scol_ref, srow_ref, nid_ref):
    ones = jnp.ones((TILE, 1), jnp.float32)
    for i in range(GRID):
        si = scol_ref[i * TILE:(i + 1) * TILE, :]
        acc = jnp.zeros((TILE, 1), jnp.float32)
        for t in range(GRID):
            sj = srow_ref[:, t * TILE:(t + 1) * TILE]
            if t < i:
                m = sj >= si
            elif t > i:
                m = sj > si
            else:
                iidx = lax.broadcasted_iota(jnp.int32, (TILE, 1), 0)
                jidx = lax.broadcasted_iota(jnp.int32, (1, TILE), 1)
                m = (sj > si) | ((sj == si) & (jidx < iidx))
            acc = acc + lax.dot_general(m.astype(jnp.float32), ones,
                                        (((1,), (0,)), ((), ())),
                                        preferred_element_type=jnp.float32)
        rank = acc.astype(jnp.int32)
        nid_ref[i * TILE:(i + 1) * TILE, :] = jnp.where(rank < K, rank, -1)


def _sc_body(newid_hbm, feats_hbm, esrc_hbm, edst_hbm,
             idx_hbm, h_hbm, osrc_hbm, odst_hbm,
             table_v, src_v, dst_v, rsrc_v, rdst_v,
             gidx_v, tgt_v, vals16_v, rows_v,
             tsem, esem, gsem0, gsem1, ssem0, ssem1, isem):
    cid = lax.axis_index("c")
    sid = lax.axis_index("s")
    wid = sid * NC + cid
    ebase = wid * EW

    tcp = pltpu.async_copy(newid_hbm, table_v, tsem)
    ecp1 = pltpu.async_copy(esrc_hbm.at[pl.ds(ebase, EW)],
                            src_v.at[pl.ds(0, EW)], esem)
    ecp2 = pltpu.async_copy(edst_hbm.at[pl.ds(ebase, EW)],
                            dst_v.at[pl.ds(0, EW)], esem)
    zeros16 = jnp.zeros((16,), jnp.int32)
    src_v[pl.ds(EWP - 16, 16)] = zeros16
    dst_v[pl.ds(EWP - 16, 16)] = zeros16
    tcp.wait()

    nbase = wid * NODES_W
    dummy = K + wid
    lane = lax.iota(jnp.int32, 16)
    zlane = jnp.zeros((16,), jnp.int32)
    for ci in range(NCH):
        base = nbase + ci * CH
        for t in range(CH // 16):
            boff = pl.multiple_of(base + t * 16, 16)
            nid = table_v[pl.ds(boff, 16)]
            lidx = boff + lane
            sel = nid >= 0
            gidx_v[ci, pl.ds(t * 16, 16)] = jnp.minimum(lidx, N - 1)
            tgt_v[ci, pl.ds(t * 16, 16)] = jnp.where(sel, nid, dummy)
            plsc.store_scatter(vals16_v, [zlane + ci, t * 16 + lane, zlane],
                               lidx)

    idescs = [pltpu.async_copy(vals16_v.at[ci], idx_hbm.at[tgt_v.at[ci]],
                               isem) for ci in range(NCH)]
    gsems = (gsem0, gsem1)
    ssems = (ssem0, ssem1)
    gd = {}
    sd = {}
    for ci in range(2):
        gd[ci] = pltpu.async_copy(feats_hbm.at[gidx_v.at[ci]],
                                  rows_v.at[ci], gsems[ci])

    ecp1.wait()
    ecp2.wait()

    @plsc.parallel_loop(0, EWP // 16, 1, unroll=8)
    def ebody(t):
        off = pl.multiple_of(t * 16, 16)
        s16 = src_v[pl.ds(off, 16)]
        d16 = dst_v[pl.ds(off, 16)]
        a = plsc.load_gather(table_v, [s16])
        b2 = plsc.load_gather(table_v, [d16])
        valid = (a >= 0) & (b2 >= 0)
        rsrc_v[pl.ds(off, 16)] = jnp.where(valid, a, -1)
        rdst_v[pl.ds(off, 16)] = jnp.where(valid, b2, -1)
    eout1 = pltpu.async_copy(rsrc_v.at[pl.ds(0, EW)],
                             osrc_hbm.at[pl.ds(ebase, EW)], esem)
    eout2 = pltpu.async_copy(rdst_v.at[pl.ds(0, EW)],
                             odst_hbm.at[pl.ds(ebase, EW)], esem)

    for ci in range(NCH):
        p = ci % 2
        gd[ci].wait()
        sd[ci] = pltpu.async_copy(rows_v.at[p], h_hbm.at[tgt_v.at[ci]],
                                  ssems[p])
        if ci + 2 < NCH:
            sd[ci].wait()
            gd[ci + 2] = pltpu.async_copy(feats_hbm.at[gidx_v.at[ci + 2]],
                                          rows_v.at[p], gsems[p])
    for ci in range(NCH - 2, NCH):
        sd[ci].wait()
    for dsc in idescs:
        dsc.wait()
    eout1.wait()
    eout2.wait()


@jax.jit
def kernel(node_features, edge_index, W, b):
    del b
    srow = pl.pallas_call(
        _score_body,
        grid=(GRID,),
        in_specs=[
            pl.BlockSpec((TILE, D), lambda i: (i, 0)),
            pl.BlockSpec((1, D), lambda i: (0, 0)),
        ],
        out_specs=pl.BlockSpec((1, TILE), lambda i: (0, i)),
        out_shape=jax.ShapeDtypeStruct((1, NP), jnp.float32),
    )(node_features, W)
    scol = jnp.reshape(srow, (NP, 1))

    new_id2 = pl.pallas_call(
        _rank_body,
        in_specs=[
            pl.BlockSpec((NP, 1), lambda: (0, 0)),
            pl.BlockSpec((1, NP), lambda: (0, 0)),
        ],
        out_specs=pl.BlockSpec((NP, 1), lambda: (0, 0)),
        out_shape=jax.ShapeDtypeStruct((NP, 1), jnp.int32),
    )(scol, srow)
    new_id = jnp.reshape(new_id2, (NP,))

    sc = pl.kernel(
        _sc_body,
        out_type=(
            jax.ShapeDtypeStruct((KP, 128), jnp.int32),
            jax.ShapeDtypeStruct((KP, D), jnp.float32),
            jax.ShapeDtypeStruct((E,), jnp.int32),
            jax.ShapeDtypeStruct((E,), jnp.int32),
        ),
        mesh=plsc.VectorSubcoreMesh(
            core_axis_name="c", subcore_axis_name="s",
            num_cores=NC, num_subcores=NS),
        compiler_params=pltpu.CompilerParams(needs_layout_passes=False),
        scratch_types=[
            pltpu.VMEM((NP,), jnp.int32),
            pltpu.VMEM((EWP,), jnp.int32),
            pltpu.VMEM((EWP,), jnp.int32),
            pltpu.VMEM((EWP,), jnp.int32),
            pltpu.VMEM((EWP,), jnp.int32),
            pltpu.VMEM((NCH, CH), jnp.int32),
            pltpu.VMEM((NCH, CH), jnp.int32),
            pltpu.VMEM((NCH, CH, 128), jnp.int32),
            pltpu.VMEM((2, CH, D), jnp.float32),
            pltpu.SemaphoreType.DMA,
            pltpu.SemaphoreType.DMA,
            pltpu.SemaphoreType.DMA,
            pltpu.SemaphoreType.DMA,
            pltpu.SemaphoreType.DMA,
            pltpu.SemaphoreType.DMA,
            pltpu.SemaphoreType.DMA,
        ],
    )
    idx_pad, h_pad, osrc, odst = sc(new_id, node_features,
                                    edge_index[0], edge_index[1])
    return (h_pad[:K], idx_pad[:K, 0], jnp.stack([osrc, odst]))


# speedup vs baseline: 1.5553x; 1.1285x over previous
import functools

import jax
import jax.numpy as jnp
from jax import lax
from jax.experimental import pallas as pl
from jax.experimental.pallas import tpu as pltpu
from jax.experimental.pallas import tpu_sc as plsc

N = 10000
D = 256
E = 160000
K = N // 2
TILE = 1024
NP = 10240
GRID = NP // TILE
KP = 5120

NC, NS = 2, 16
NW = NC * NS
NODES_W = NP // NW
CH = 64
NCH = NODES_W // CH
EW = E // NW
EWP = 5008
NEG_INF = float("-inf")


def _score_body(x_ref, w_ref, srow_ref, scol_ref):
    i = pl.program_id(0)
    x = x_ref[...]
    w = w_ref[...]
    row = lax.dot_general(w, x, (((1,), (1,)), ((), ())),
                          preferred_element_type=jnp.float32)
    cidx = i * TILE + lax.broadcasted_iota(jnp.int32, (1, TILE), 1)
    rowm = jnp.where(cidx < N, row, NEG_INF)
    srow_ref[...] = rowm
    scol_ref[...] = jnp.transpose(rowm, (1, 0))


def _rank_body(scol_ref, srow_ref, nid_ref):
    ones = jnp.ones((TILE, 1), jnp.float32)
    for i in range(GRID):
        si = scol_ref[i * TILE:(i + 1) * TILE, :]
        acc = jnp.zeros((TILE, 1), jnp.float32)
        for t in range(GRID):
            sj = srow_ref[:, t * TILE:(t + 1) * TILE]
            if t < i:
                m = sj >= si
            elif t > i:
                m = sj > si
            else:
                iidx = lax.broadcasted_iota(jnp.int32, (TILE, 1), 0)
                jidx = lax.broadcasted_iota(jnp.int32, (1, TILE), 1)
                m = (sj > si) | ((sj == si) & (jidx < iidx))
            acc = acc + lax.dot_general(m.astype(jnp.float32), ones,
                                        (((1,), (0,)), ((), ())),
                                        preferred_element_type=jnp.float32)
        rank = acc.astype(jnp.int32)
        nid = jnp.where(rank < K, rank, -1)
        nid_ref[i * TILE:(i + 1) * TILE] = jnp.reshape(nid, (TILE,))


def _sc_body(newid_hbm, feats_hbm, edges_hbm,
             idx_hbm, h_hbm, sub_hbm,
             table_v, src_v, dst_v, rsrc_v, rdst_v,
             gidx_v, tgt_v, vals16_v, rows_v,
             tsem, esem, gsem0, gsem1, ssem0, ssem1, isem):
    cid = lax.axis_index("c")
    sid = lax.axis_index("s")
    wid = sid * NC + cid
    ebase = wid * EW

    tcp = pltpu.async_copy(newid_hbm, table_v, tsem)
    ecp1 = pltpu.async_copy(edges_hbm.at[pl.ds(ebase, EW)],
                            src_v.at[pl.ds(0, EW)], esem)
    ecp2 = pltpu.async_copy(edges_hbm.at[pl.ds(E + ebase, EW)],
                            dst_v.at[pl.ds(0, EW)], esem)
    zeros16 = jnp.zeros((16,), jnp.int32)
    src_v[pl.ds(EWP - 16, 16)] = zeros16
    dst_v[pl.ds(EWP - 16, 16)] = zeros16
    tcp.wait()

    nbase = wid * NODES_W
    dummy = K + wid
    lane = lax.iota(jnp.int32, 16)
    zlane = jnp.zeros((16,), jnp.int32)
    for ci in range(NCH):
        base = nbase + ci * CH
        for t in range(CH // 16):
            boff = pl.multiple_of(base + t * 16, 16)
            nid = table_v[pl.ds(boff, 16)]
            lidx = boff + lane
            sel = nid >= 0
            gidx_v[ci, pl.ds(t * 16, 16)] = jnp.minimum(lidx, N - 1)
            tgt_v[ci, pl.ds(t * 16, 16)] = jnp.where(sel, nid, dummy)
            plsc.store_scatter(vals16_v, [zlane + ci, t * 16 + lane, zlane],
                               lidx)

    idescs = [pltpu.async_copy(vals16_v.at[ci], idx_hbm.at[tgt_v.at[ci]],
                               isem) for ci in range(NCH)]
    gsems = (gsem0, gsem1)
    ssems = (ssem0, ssem1)
    gd = {}
    sd = {}
    for ci in range(2):
        gd[ci] = pltpu.async_copy(feats_hbm.at[gidx_v.at[ci]],
                                  rows_v.at[ci], gsems[ci])

    ecp1.wait()
    ecp2.wait()

    @plsc.parallel_loop(0, EWP // 16, 1, unroll=8)
    def ebody(t):
        off = pl.multiple_of(t * 16, 16)
        s16 = src_v[pl.ds(off, 16)]
        d16 = dst_v[pl.ds(off, 16)]
        a = plsc.load_gather(table_v, [s16])
        b2 = plsc.load_gather(table_v, [d16])
        valid = (a >= 0) & (b2 >= 0)
        rsrc_v[pl.ds(off, 16)] = jnp.where(valid, a, -1)
        rdst_v[pl.ds(off, 16)] = jnp.where(valid, b2, -1)
    eout1 = pltpu.async_copy(rsrc_v.at[pl.ds(0, EW)],
                             sub_hbm.at[pl.ds(ebase, EW)], esem)
    eout2 = pltpu.async_copy(rdst_v.at[pl.ds(0, EW)],
                             sub_hbm.at[pl.ds(E + ebase, EW)], esem)

    for ci in range(NCH):
        p = ci % 2
        gd[ci].wait()
        sd[ci] = pltpu.async_copy(rows_v.at[p], h_hbm.at[tgt_v.at[ci]],
                                  ssems[p])
        if ci + 2 < NCH:
            sd[ci].wait()
            gd[ci + 2] = pltpu.async_copy(feats_hbm.at[gidx_v.at[ci + 2]],
                                          rows_v.at[p], gsems[p])
    for ci in range(NCH - 2, NCH):
        sd[ci].wait()
    for dsc in idescs:
        dsc.wait()
    eout1.wait()
    eout2.wait()


@jax.jit
def kernel(node_features, edge_index, W, b):
    del b
    srow, scol = pl.pallas_call(
        _score_body,
        grid=(GRID,),
        in_specs=[
            pl.BlockSpec((TILE, D), lambda i: (i, 0)),
            pl.BlockSpec((1, D), lambda i: (0, 0)),
        ],
        out_specs=[
            pl.BlockSpec((1, TILE), lambda i: (0, i)),
            pl.BlockSpec((TILE, 1), lambda i: (i, 0)),
        ],
        out_shape=[
            jax.ShapeDtypeStruct((1, NP), jnp.float32),
            jax.ShapeDtypeStruct((NP, 1), jnp.float32),
        ],
    )(node_features, W)

    new_id2 = pl.pallas_call(
        _rank_body,
        in_specs=[
            pl.BlockSpec((NP, 1), lambda: (0, 0)),
            pl.BlockSpec((1, NP), lambda: (0, 0)),
        ],
        out_specs=pl.BlockSpec((NP,), lambda: (0,)),
        out_shape=jax.ShapeDtypeStruct((NP,), jnp.int32),
    )(scol, srow)
    new_id = new_id2

    sc = pl.kernel(
        _sc_body,
        out_type=(
            jax.ShapeDtypeStruct((KP, 128), jnp.int32),
            jax.ShapeDtypeStruct((KP, D), jnp.float32),
            jax.ShapeDtypeStruct((2 * E,), jnp.int32),
        ),
        mesh=plsc.VectorSubcoreMesh(
            core_axis_name="c", subcore_axis_name="s",
            num_cores=NC, num_subcores=NS),
        compiler_params=pltpu.CompilerParams(needs_layout_passes=False),
        scratch_types=[
            pltpu.VMEM((NP,), jnp.int32),
            pltpu.VMEM((EWP,), jnp.int32),
            pltpu.VMEM((EWP,), jnp.int32),
            pltpu.VMEM((EWP,), jnp.int32),
            pltpu.VMEM((EWP,), jnp.int32),
            pltpu.VMEM((NCH, CH), jnp.int32),
            pltpu.VMEM((NCH, CH), jnp.int32),
            pltpu.VMEM((NCH, CH, 128), jnp.int32),
            pltpu.VMEM((2, CH, D), jnp.float32),
            pltpu.SemaphoreType.DMA,
            pltpu.SemaphoreType.DMA,
            pltpu.SemaphoreType.DMA,
            pltpu.SemaphoreType.DMA,
            pltpu.SemaphoreType.DMA,
            pltpu.SemaphoreType.DMA,
            pltpu.SemaphoreType.DMA,
        ],
    )
    idx_pad, h_pad, sub = sc(new_id, node_features,
                             jnp.reshape(edge_index, (2 * E,)))
    return (h_pad[:K], idx_pad[:K, 0], jnp.reshape(sub, (2, E)))


# speedup vs baseline: 2.0422x; 1.3130x over previous
import functools

import jax
import jax.numpy as jnp
from jax import lax
from jax.experimental import pallas as pl
from jax.experimental.pallas import tpu as pltpu
from jax.experimental.pallas import tpu_sc as plsc

N = 10000
D = 256
E = 160000
K = N // 2
TILE = 1024
NP = 10240
GRID = NP // TILE
KP = 5120

NC, NS = 2, 16
NW = NC * NS
NODES_W = NP // NW
HW = KP // NW
HCH = 80
NHCH = HW // HCH
EW = E // NW
EWP = 5008
NEG_INF = float("-inf")


def _score_body(x_ref, w_ref, srow_ref, scol_ref):
    i = pl.program_id(0)
    x = x_ref[...]
    w = w_ref[...]
    row = lax.dot_general(w, x, (((1,), (1,)), ((), ())),
                          preferred_element_type=jnp.float32)
    cidx = i * TILE + lax.broadcasted_iota(jnp.int32, (1, TILE), 1)
    rowm = jnp.where(cidx < N, row, NEG_INF)
    srow_ref[...] = rowm
    scol_ref[...] = jnp.transpose(rowm, (1, 0))


def _rank_body(scol_ref, srow_ref, nid_ref):
    ones_c = jnp.ones((TILE, 1), jnp.bfloat16)
    ones_r = jnp.ones((1, TILE), jnp.bfloat16)
    acc_col = [jnp.zeros((TILE, 1), jnp.float32) for _ in range(GRID)]
    acc_row = [jnp.zeros((1, TILE), jnp.float32) for _ in range(GRID)]
    for a in range(GRID):
        sa = scol_ref[a * TILE:(a + 1) * TILE, :]
        for b in range(a + 1, GRID):
            sb = srow_ref[:, b * TILE:(b + 1) * TILE]
            m = (sb > sa).astype(jnp.bfloat16)
            acc_col[a] = acc_col[a] + lax.dot_general(
                m, ones_c, (((1,), (0,)), ((), ())),
                preferred_element_type=jnp.float32)
            acc_row[b] = acc_row[b] + lax.dot_general(
                ones_r, m, (((1,), (0,)), ((), ())),
                preferred_element_type=jnp.float32)
    iidx = lax.broadcasted_iota(jnp.int32, (TILE, 1), 0)
    jidx = lax.broadcasted_iota(jnp.int32, (1, TILE), 1)
    for a in range(GRID):
        sa = scol_ref[a * TILE:(a + 1) * TILE, :]
        sr = srow_ref[:, a * TILE:(a + 1) * TILE]
        md = ((sr > sa) | ((sr == sa) & (jidx < iidx))).astype(jnp.bfloat16)
        acc_col[a] = acc_col[a] + lax.dot_general(
            md, ones_c, (((1,), (0,)), ((), ())),
            preferred_element_type=jnp.float32)
    for a in range(GRID):
        rank_f = (acc_col[a] + float(TILE * a)
                  - jnp.transpose(acc_row[a], (1, 0)))
        rank = rank_f.astype(jnp.int32)
        nid = jnp.where(rank < K, rank, -1)
        nid_ref[a * TILE:(a + 1) * TILE] = jnp.reshape(nid, (TILE,))


def _sc_body(newid_hbm, feats_hbm, edges_hbm,
             idx_hbm, h_hbm, sub_hbm,
             table_v, idxloc_v, src_v, dst_v, rsrc_v, rdst_v,
             gidx_v, rows_v,
             tsem, esem, xsem, gsem0, gsem1, wsem0, wsem1):
    cid = lax.axis_index("c")
    sid = lax.axis_index("s")
    wid = sid * NC + cid
    ebase = wid * EW

    tcp = pltpu.async_copy(newid_hbm, table_v, tsem)
    ecp1 = pltpu.async_copy(edges_hbm.at[pl.ds(ebase, EW)],
                            src_v.at[pl.ds(0, EW)], esem)
    ecp2 = pltpu.async_copy(edges_hbm.at[pl.ds(E + ebase, EW)],
                            dst_v.at[pl.ds(0, EW)], esem)
    zeros16 = jnp.zeros((16,), jnp.int32)
    src_v[pl.ds(EWP - 16, 16)] = zeros16
    dst_v[pl.ds(EWP - 16, 16)] = zeros16
    lane = lax.iota(jnp.int32, 16)
    tcp.wait()

    @plsc.parallel_loop(0, NP // 16, 1, unroll=8)
    def invbody(t):
        off = pl.multiple_of(t * 16, 16)
        nid = table_v[pl.ds(off, 16)]
        tgt = jnp.where(nid >= 0, nid, KP)
        plsc.store_scatter(idxloc_v, [tgt], off + lane)

    ibase = wid * HW
    xcp = pltpu.async_copy(idxloc_v.at[pl.ds(ibase, HW)],
                           idx_hbm.at[pl.ds(ibase, HW)], xsem)

    for c2 in range(HW // 16):
        off = pl.multiple_of(ibase + c2 * 16, 16)
        v = idxloc_v[pl.ds(off, 16)]
        gidx_v[c2 // (HCH // 16), pl.ds((c2 % (HCH // 16)) * 16, 16)] = (
            jnp.minimum(jnp.maximum(v, 0), N - 1))
    gsems = (gsem0, gsem1)
    wsems = (wsem0, wsem1)
    gd = {}
    for c in range(NHCH):
        gd[c] = pltpu.async_copy(feats_hbm.at[gidx_v.at[c]],
                                 rows_v.at[c], gsems[c])

    ecp1.wait()
    ecp2.wait()

    @plsc.parallel_loop(0, EWP // 16, 1, unroll=8)
    def ebody(t):
        off = pl.multiple_of(t * 16, 16)
        s16 = src_v[pl.ds(off, 16)]
        d16 = dst_v[pl.ds(off, 16)]
        a = plsc.load_gather(table_v, [s16])
        b2 = plsc.load_gather(table_v, [d16])
        valid = (a >= 0) & (b2 >= 0)
        rsrc_v[pl.ds(off, 16)] = jnp.where(valid, a, -1)
        rdst_v[pl.ds(off, 16)] = jnp.where(valid, b2, -1)
    eout1 = pltpu.async_copy(rsrc_v.at[pl.ds(0, EW)],
                             sub_hbm.at[pl.ds(ebase, EW)], esem)
    eout2 = pltpu.async_copy(rdst_v.at[pl.ds(0, EW)],
                             sub_hbm.at[pl.ds(E + ebase, EW)], esem)

    wd = {}
    for c in range(NHCH):
        gd[c].wait()
        wd[c] = pltpu.async_copy(rows_v.at[c],
                                 h_hbm.at[pl.ds(ibase + c * HCH, HCH)],
                                 wsems[c])
    for c in range(NHCH):
        wd[c].wait()
    xcp.wait()
    eout1.wait()
    eout2.wait()


@jax.jit
def kernel(node_features, edge_index, W, b):
    del b
    srow, scol = pl.pallas_call(
        _score_body,
        grid=(GRID,),
        in_specs=[
            pl.BlockSpec((TILE, D), lambda i: (i, 0)),
            pl.BlockSpec((1, D), lambda i: (0, 0)),
        ],
        out_specs=[
            pl.BlockSpec((1, TILE), lambda i: (0, i)),
            pl.BlockSpec((TILE, 1), lambda i: (i, 0)),
        ],
        out_shape=[
            jax.ShapeDtypeStruct((1, NP), jnp.float32),
            jax.ShapeDtypeStruct((NP, 1), jnp.float32),
        ],
    )(node_features, W)

    new_id2 = pl.pallas_call(
        _rank_body,
        in_specs=[
            pl.BlockSpec((NP, 1), lambda: (0, 0)),
            pl.BlockSpec((1, NP), lambda: (0, 0)),
        ],
        out_specs=pl.BlockSpec((NP,), lambda: (0,)),
        out_shape=jax.ShapeDtypeStruct((NP,), jnp.int32),
    )(scol, srow)
    new_id = new_id2

    sc = pl.kernel(
        _sc_body,
        out_type=(
            jax.ShapeDtypeStruct((KP,), jnp.int32),
            jax.ShapeDtypeStruct((KP, D), jnp.float32),
            jax.ShapeDtypeStruct((2 * E,), jnp.int32),
        ),
        mesh=plsc.VectorSubcoreMesh(
            core_axis_name="c", subcore_axis_name="s",
            num_cores=NC, num_subcores=NS),
        compiler_params=pltpu.CompilerParams(needs_layout_passes=False),
        scratch_types=[
            pltpu.VMEM((NP,), jnp.int32),
            pltpu.VMEM((KP + 16,), jnp.int32),
            pltpu.VMEM((EWP,), jnp.int32),
            pltpu.VMEM((EWP,), jnp.int32),
            pltpu.VMEM((EWP,), jnp.int32),
            pltpu.VMEM((EWP,), jnp.int32),
            pltpu.VMEM((NHCH, HCH), jnp.int32),
            pltpu.VMEM((NHCH, HCH, D), jnp.float32),
            pltpu.SemaphoreType.DMA,
            pltpu.SemaphoreType.DMA,
            pltpu.SemaphoreType.DMA,
            pltpu.SemaphoreType.DMA,
            pltpu.SemaphoreType.DMA,
            pltpu.SemaphoreType.DMA,
            pltpu.SemaphoreType.DMA,
        ],
    )
    idx_pad, h_pad, sub = sc(new_id, node_features,
                             jnp.reshape(edge_index, (2 * E,)))
    return (h_pad[:K], idx_pad[:K], jnp.reshape(sub, (2, E)))


# speedup vs baseline: 2.2257x; 1.0898x over previous
import functools

import jax
import jax.numpy as jnp
from jax import lax
from jax.experimental import pallas as pl
from jax.experimental.pallas import tpu as pltpu
from jax.experimental.pallas import tpu_sc as plsc

N = 10000
D = 256
E = 160000
K = N // 2
TILE = 1024
NP = 10240
GRID = NP // TILE
KP = 5120

NC, NS = 2, 16
NW = NC * NS
NODES_W = NP // NW
HW = KP // NW
HTAIL = K - (NW - 1) * HW
HCH = 80
NHCH = HW // HCH
EW = E // NW
EWP = 5008
NEG_INF = float("-inf")


def _score_body(x_ref, w_ref, srow_ref, scol_ref):
    i = pl.program_id(0)
    x = x_ref[...]
    w = w_ref[...]
    row = lax.dot_general(w, x, (((1,), (1,)), ((), ())),
                          preferred_element_type=jnp.float32)
    cidx = i * TILE + lax.broadcasted_iota(jnp.int32, (1, TILE), 1)
    rowm = jnp.where(cidx < N, row, NEG_INF)
    srow_ref[...] = rowm
    scol_ref[...] = jnp.transpose(rowm, (1, 0))


def _rank_body(scol_ref, srow_ref, nid_ref):
    ones_col = jnp.ones((NP, 1), jnp.bfloat16)
    ones_row = jnp.ones((1, TILE), jnp.bfloat16)
    acc_col = [jnp.zeros((TILE, 1), jnp.float32) for _ in range(GRID)]
    acc_row = [jnp.zeros((1, TILE), jnp.float32) for _ in range(GRID)]
    WCHUNK = 4
    for a in range(GRID - 1):
        sa = scol_ref[a * TILE:(a + 1) * TILE, :]
        for b0 in range(a + 1, GRID, WCHUNK):
            bw = min(WCHUNK, GRID - b0)
            X = bw * TILE
            sj = srow_ref[:, b0 * TILE:b0 * TILE + X]
            m = (sj > sa).astype(jnp.bfloat16)
            acc_col[a] = acc_col[a] + lax.dot_general(
                m, ones_col[:X], (((1,), (0,)), ((), ())),
                preferred_element_type=jnp.float32)
            csum = lax.dot_general(ones_row, m, (((1,), (0,)), ((), ())),
                                   preferred_element_type=jnp.float32)
            for j in range(bw):
                acc_row[b0 + j] = acc_row[b0 + j] + csum[:, j * TILE:
                                                         (j + 1) * TILE]
    iidx = lax.broadcasted_iota(jnp.int32, (TILE, 1), 0)
    jidx = lax.broadcasted_iota(jnp.int32, (1, TILE), 1)
    for a in range(GRID):
        sa = scol_ref[a * TILE:(a + 1) * TILE, :]
        sr = srow_ref[:, a * TILE:(a + 1) * TILE]
        md = ((sr > sa) | ((sr == sa) & (jidx < iidx))).astype(jnp.bfloat16)
        acc_col[a] = acc_col[a] + lax.dot_general(
            md, ones_col[:TILE], (((1,), (0,)), ((), ())),
            preferred_element_type=jnp.float32)
    for a in range(GRID):
        rank_f = (acc_col[a] + float(TILE * a)
                  - jnp.transpose(acc_row[a], (1, 0)))
        rank = rank_f.astype(jnp.int32)
        nid = jnp.where(rank < K, rank, -1)
        nid_ref[a * TILE:(a + 1) * TILE] = jnp.reshape(nid, (TILE,))


def _sc_body(newid_hbm, feats_hbm, edges_hbm,
             idx_hbm, h_hbm, sub_hbm,
             table_v, idxloc_v, src_v, dst_v, rsrc_v, rdst_v,
             gidx_v, rows_v,
             tsem, esem, xsem, gsem0, gsem1, wsem0, wsem1):
    cid = lax.axis_index("c")
    sid = lax.axis_index("s")
    wid = sid * NC + cid
    ebase = wid * EW

    tcp = pltpu.async_copy(newid_hbm, table_v, tsem)
    ecp1 = pltpu.async_copy(edges_hbm.at[pl.ds(ebase, EW)],
                            src_v.at[pl.ds(0, EW)], esem)
    ecp2 = pltpu.async_copy(edges_hbm.at[pl.ds(E + ebase, EW)],
                            dst_v.at[pl.ds(0, EW)], esem)
    zeros16 = jnp.zeros((16,), jnp.int32)
    src_v[pl.ds(EWP - 16, 16)] = zeros16
    dst_v[pl.ds(EWP - 16, 16)] = zeros16
    lane = lax.iota(jnp.int32, 16)
    tcp.wait()

    @plsc.parallel_loop(0, NP // 16, 1, unroll=8)
    def invbody(t):
        off = pl.multiple_of(t * 16, 16)
        nid = table_v[pl.ds(off, 16)]
        tgt = jnp.where(nid >= 0, nid, KP)
        plsc.store_scatter(idxloc_v, [tgt], off + lane)

    ibase = wid * HW

    @pl.when(wid < NW - 1)
    def _():
        pltpu.async_copy(idxloc_v.at[pl.ds(ibase, HW)],
                         idx_hbm.at[pl.ds(ibase, HW)], xsem).wait()

    @pl.when(wid == NW - 1)
    def _():
        pltpu.async_copy(idxloc_v.at[pl.ds(ibase, HTAIL)],
                         idx_hbm.at[pl.ds(ibase, HTAIL)], xsem).wait()

    for c2 in range(HW // 16):
        off = pl.multiple_of(ibase + c2 * 16, 16)
        v = idxloc_v[pl.ds(off, 16)]
        gidx_v[c2 // (HCH // 16), pl.ds((c2 % (HCH // 16)) * 16, 16)] = (
            jnp.minimum(jnp.maximum(v, 0), N - 1))
    gsems = (gsem0, gsem1)
    wsems = (wsem0, wsem1)
    gd = {}
    for c in range(NHCH):
        gd[c] = pltpu.async_copy(feats_hbm.at[gidx_v.at[c]],
                                 rows_v.at[c], gsems[c])

    ecp1.wait()
    ecp2.wait()

    @plsc.parallel_loop(0, EWP // 16, 1, unroll=8)
    def ebody(t):
        off = pl.multiple_of(t * 16, 16)
        s16 = src_v[pl.ds(off, 16)]
        d16 = dst_v[pl.ds(off, 16)]
        a = plsc.load_gather(table_v, [s16])
        b2 = plsc.load_gather(table_v, [d16])
        valid = (a >= 0) & (b2 >= 0)
        rsrc_v[pl.ds(off, 16)] = jnp.where(valid, a, -1)
        rdst_v[pl.ds(off, 16)] = jnp.where(valid, b2, -1)
    eout1 = pltpu.async_copy(rsrc_v.at[pl.ds(0, EW)],
                             sub_hbm.at[pl.ds(ebase, EW)], esem)
    eout2 = pltpu.async_copy(rdst_v.at[pl.ds(0, EW)],
                             sub_hbm.at[pl.ds(E + ebase, EW)], esem)

    for c in range(NHCH):
        gd[c].wait()

    @pl.when(wid < NW - 1)
    def _():
        wd = [pltpu.async_copy(rows_v.at[c],
                               h_hbm.at[pl.ds(ibase + c * HCH, HCH)],
                               wsems[c]) for c in range(NHCH)]
        for d in wd:
            d.wait()

    @pl.when(wid == NW - 1)
    def _():
        pltpu.async_copy(rows_v.at[0, pl.ds(0, HTAIL)],
                         h_hbm.at[pl.ds(ibase, HTAIL)], wsem0).wait()

    eout1.wait()
    eout2.wait()


@jax.jit
def kernel(node_features, edge_index, W, b):
    del b
    srow, scol = pl.pallas_call(
        _score_body,
        grid=(GRID,),
        in_specs=[
            pl.BlockSpec((TILE, D), lambda i: (i, 0)),
            pl.BlockSpec((1, D), lambda i: (0, 0)),
        ],
        out_specs=[
            pl.BlockSpec((1, TILE), lambda i: (0, i)),
            pl.BlockSpec((TILE, 1), lambda i: (i, 0)),
        ],
        out_shape=[
            jax.ShapeDtypeStruct((1, NP), jnp.float32),
            jax.ShapeDtypeStruct((NP, 1), jnp.float32),
        ],
    )(node_features, W)

    new_id2 = pl.pallas_call(
        _rank_body,
        in_specs=[
            pl.BlockSpec((NP, 1), lambda: (0, 0)),
            pl.BlockSpec((1, NP), lambda: (0, 0)),
        ],
        out_specs=pl.BlockSpec((NP,), lambda: (0,)),
        out_shape=jax.ShapeDtypeStruct((NP,), jnp.int32),
    )(scol, srow)
    new_id = new_id2

    sc = pl.kernel(
        _sc_body,
        out_type=(
            jax.ShapeDtypeStruct((K,), jnp.int32),
            jax.ShapeDtypeStruct((K, D), jnp.float32),
            jax.ShapeDtypeStruct((2 * E,), jnp.int32),
        ),
        mesh=plsc.VectorSubcoreMesh(
            core_axis_name="c", subcore_axis_name="s",
            num_cores=NC, num_subcores=NS),
        compiler_params=pltpu.CompilerParams(needs_layout_passes=False),
        scratch_types=[
            pltpu.VMEM((NP,), jnp.int32),
            pltpu.VMEM((KP + 16,), jnp.int32),
            pltpu.VMEM((EWP,), jnp.int32),
            pltpu.VMEM((EWP,), jnp.int32),
            pltpu.VMEM((EWP,), jnp.int32),
            pltpu.VMEM((EWP,), jnp.int32),
            pltpu.VMEM((NHCH, HCH), jnp.int32),
            pltpu.VMEM((NHCH, HCH, D), jnp.float32),
            pltpu.SemaphoreType.DMA,
            pltpu.SemaphoreType.DMA,
            pltpu.SemaphoreType.DMA,
            pltpu.SemaphoreType.DMA,
            pltpu.SemaphoreType.DMA,
            pltpu.SemaphoreType.DMA,
            pltpu.SemaphoreType.DMA,
        ],
    )
    idx_out, h_out, sub = sc(new_id, node_features,
                             jnp.reshape(edge_index, (2 * E,)))
    return (h_out, idx_out, jnp.reshape(sub, (2, E)))


# speedup vs baseline: 2.4222x; 1.0883x over previous
import functools

import jax
import jax.numpy as jnp
from jax import lax
from jax.experimental import pallas as pl
from jax.experimental.pallas import tpu as pltpu
from jax.experimental.pallas import tpu_sc as plsc

N = 10000
D = 256
E = 160000
K = N // 2
TILE = 1024
NP = 10240
GRID = NP // TILE
KP = 5120

NC, NS = 2, 16
NW = NC * NS
NODES_W = NP // NW
HW = KP // NW
HTAIL = K - (NW - 1) * HW
HCH = 80
NHCH = HW // HCH
EW = E // NW
EWP = 5008
NEG_INF = float("-inf")


def _rank_body(x_ref, w_ref, nid_ref, srow_ref, scol_ref):
    w = w_ref[...]
    for i in range(GRID):
        lo = i * TILE
        hi = min((i + 1) * TILE, N)
        x = x_ref[lo:hi, :]
        row = lax.dot_general(w, x, (((1,), (1,)), ((), ())),
                              preferred_element_type=jnp.float32)
        if hi - lo < TILE:
            row = jnp.concatenate(
                [row, jnp.full((1, TILE - (hi - lo)), NEG_INF, jnp.float32)],
                axis=1)
        srow_ref[:, lo:lo + TILE] = row
        scol_ref[lo:lo + TILE, :] = jnp.transpose(row, (1, 0))
    ones_col = jnp.ones((NP, 1), jnp.bfloat16)
    ones_row = jnp.ones((1, TILE), jnp.bfloat16)
    acc_col = [jnp.zeros((TILE, 1), jnp.float32) for _ in range(GRID)]
    acc_row = [jnp.zeros((1, TILE), jnp.float32) for _ in range(GRID)]
    WCHUNK = 4
    for a in range(GRID - 1):
        sa = scol_ref[a * TILE:(a + 1) * TILE, :]
        for b0 in range(a + 1, GRID, WCHUNK):
            bw = min(WCHUNK, GRID - b0)
            X = bw * TILE
            sj = srow_ref[:, b0 * TILE:b0 * TILE + X]
            m = (sj > sa).astype(jnp.bfloat16)
            acc_col[a] = acc_col[a] + lax.dot_general(
                m, ones_col[:X], (((1,), (0,)), ((), ())),
                preferred_element_type=jnp.float32)
            csum = lax.dot_general(ones_row, m, (((1,), (0,)), ((), ())),
                                   preferred_element_type=jnp.float32)
            for j in range(bw):
                acc_row[b0 + j] = acc_row[b0 + j] + csum[:, j * TILE:
                                                         (j + 1) * TILE]
    iidx = lax.broadcasted_iota(jnp.int32, (TILE, 1), 0)
    jidx = lax.broadcasted_iota(jnp.int32, (1, TILE), 1)
    for a in range(GRID):
        sa = scol_ref[a * TILE:(a + 1) * TILE, :]
        sr = srow_ref[:, a * TILE:(a + 1) * TILE]
        md = ((sr > sa) | ((sr == sa) & (jidx < iidx))).astype(jnp.bfloat16)
        acc_col[a] = acc_col[a] + lax.dot_general(
            md, ones_col[:TILE], (((1,), (0,)), ((), ())),
            preferred_element_type=jnp.float32)
    for a in range(GRID):
        rank_f = (acc_col[a] + float(TILE * a)
                  - jnp.transpose(acc_row[a], (1, 0)))
        rank = rank_f.astype(jnp.int32)
        nid = jnp.where(rank < K, rank, -1)
        nid_ref[a * TILE:(a + 1) * TILE] = jnp.reshape(nid, (TILE,))


def _sc_body(newid_hbm, feats_hbm, edges_hbm,
             idx_hbm, h_hbm, sub_hbm,
             table_v, idxloc_v, src_v, dst_v, rsrc_v, rdst_v,
             gidx_v, rows_v,
             tsem, esem, xsem, gsem0, gsem1, wsem0, wsem1):
    cid = lax.axis_index("c")
    sid = lax.axis_index("s")
    wid = sid * NC + cid
    ebase = wid * EW

    tcp = pltpu.async_copy(newid_hbm, table_v, tsem)
    ecp1 = pltpu.async_copy(edges_hbm.at[pl.ds(ebase, EW)],
                            src_v.at[pl.ds(0, EW)], esem)
    ecp2 = pltpu.async_copy(edges_hbm.at[pl.ds(E + ebase, EW)],
                            dst_v.at[pl.ds(0, EW)], esem)
    zeros16 = jnp.zeros((16,), jnp.int32)
    src_v[pl.ds(EWP - 16, 16)] = zeros16
    dst_v[pl.ds(EWP - 16, 16)] = zeros16
    lane = lax.iota(jnp.int32, 16)
    tcp.wait()

    @plsc.parallel_loop(0, NP // 16, 1, unroll=8)
    def invbody(t):
        off = pl.multiple_of(t * 16, 16)
        nid = table_v[pl.ds(off, 16)]
        tgt = jnp.where(nid >= 0, nid, KP)
        plsc.store_scatter(idxloc_v, [tgt], off + lane)

    ibase = wid * HW

    @pl.when(wid < NW - 1)
    def _():
        pltpu.async_copy(idxloc_v.at[pl.ds(ibase, HW)],
                         idx_hbm.at[pl.ds(ibase, HW)], xsem).wait()

    @pl.when(wid == NW - 1)
    def _():
        pltpu.async_copy(idxloc_v.at[pl.ds(ibase, HTAIL)],
                         idx_hbm.at[pl.ds(ibase, HTAIL)], xsem).wait()

    for c2 in range(HW // 16):
        off = pl.multiple_of(ibase + c2 * 16, 16)
        v = idxloc_v[pl.ds(off, 16)]
        gidx_v[c2 // (HCH // 16), pl.ds((c2 % (HCH // 16)) * 16, 16)] = (
            jnp.minimum(jnp.maximum(v, 0), N - 1))
    gsems = (gsem0, gsem1)
    wsems = (wsem0, wsem1)
    gd = {}
    for c in range(NHCH):
        gd[c] = pltpu.async_copy(feats_hbm.at[gidx_v.at[c]],
                                 rows_v.at[c], gsems[c])

    ecp1.wait()
    ecp2.wait()

    @plsc.parallel_loop(0, EWP // 16, 1, unroll=8)
    def ebody(t):
        off = pl.multiple_of(t * 16, 16)
        s16 = src_v[pl.ds(off, 16)]
        d16 = dst_v[pl.ds(off, 16)]
        a = plsc.load_gather(table_v, [s16])
        b2 = plsc.load_gather(table_v, [d16])
        valid = (a >= 0) & (b2 >= 0)
        rsrc_v[pl.ds(off, 16)] = jnp.where(valid, a, -1)
        rdst_v[pl.ds(off, 16)] = jnp.where(valid, b2, -1)
    eout1 = pltpu.async_copy(rsrc_v.at[pl.ds(0, EW)],
                             sub_hbm.at[pl.ds(ebase, EW)], esem)
    eout2 = pltpu.async_copy(rdst_v.at[pl.ds(0, EW)],
                             sub_hbm.at[pl.ds(E + ebase, EW)], esem)

    for c in range(NHCH):
        gd[c].wait()

    @pl.when(wid < NW - 1)
    def _():
        wd = [pltpu.async_copy(rows_v.at[c],
                               h_hbm.at[pl.ds(ibase + c * HCH, HCH)],
                               wsems[c]) for c in range(NHCH)]
        for d in wd:
            d.wait()

    @pl.when(wid == NW - 1)
    def _():
        pltpu.async_copy(rows_v.at[0, pl.ds(0, HTAIL)],
                         h_hbm.at[pl.ds(ibase, HTAIL)], wsem0).wait()

    eout1.wait()
    eout2.wait()


@jax.jit
def kernel(node_features, edge_index, W, b):
    del b
    new_id = pl.pallas_call(
        _rank_body,
        in_specs=[
            pl.BlockSpec((N, D), lambda: (0, 0)),
            pl.BlockSpec((1, D), lambda: (0, 0)),
        ],
        out_specs=pl.BlockSpec((NP,), lambda: (0,)),
        out_shape=jax.ShapeDtypeStruct((NP,), jnp.int32),
        scratch_shapes=[
            pltpu.VMEM((1, NP), jnp.float32),
            pltpu.VMEM((NP, 1), jnp.float32),
        ],
    )(node_features, W)

    sc = pl.kernel(
        _sc_body,
        out_type=(
            jax.ShapeDtypeStruct((K,), jnp.int32),
            jax.ShapeDtypeStruct((K, D), jnp.float32),
            jax.ShapeDtypeStruct((2 * E,), jnp.int32),
        ),
        mesh=plsc.VectorSubcoreMesh(
            core_axis_name="c", subcore_axis_name="s",
            num_cores=NC, num_subcores=NS),
        compiler_params=pltpu.CompilerParams(needs_layout_passes=False),
        scratch_types=[
            pltpu.VMEM((NP,), jnp.int32),
            pltpu.VMEM((KP + 16,), jnp.int32),
            pltpu.VMEM((EWP,), jnp.int32),
            pltpu.VMEM((EWP,), jnp.int32),
            pltpu.VMEM((EWP,), jnp.int32),
            pltpu.VMEM((EWP,), jnp.int32),
            pltpu.VMEM((NHCH, HCH), jnp.int32),
            pltpu.VMEM((NHCH, HCH, D), jnp.float32),
            pltpu.SemaphoreType.DMA,
            pltpu.SemaphoreType.DMA,
            pltpu.SemaphoreType.DMA,
            pltpu.SemaphoreType.DMA,
            pltpu.SemaphoreType.DMA,
            pltpu.SemaphoreType.DMA,
            pltpu.SemaphoreType.DMA,
        ],
    )
    idx_out, h_out, sub = sc(new_id, node_features,
                             jnp.reshape(edge_index, (2 * E,)))
    return (h_out, idx_out, jnp.reshape(sub, (2, E)))
